# Initial kernel scaffold; baseline (speedup 1.0000x reference)
import jax
import jax.numpy as jnp
from jax.experimental import pallas as pl

A = 512
NB = 32
NF = 128


def _probe_body(tab_ref, n_ref, out_ref):
    tab = tab_ref[0]
    idx = n_ref[0]
    idxf = idx.reshape(A * NB, 1)
    idxb = jnp.broadcast_to(idxf, (A * NB, NF))
    g = jnp.take_along_axis(tab, idxb, axis=0)
    s = g.reshape(A, NB, NF).sum(axis=1)
    out_ref[0, 0] = jnp.sum(s)


def kernel(Z, R, N, NM, AM, emb, Wrbf, brbf, Wa1, ba1, Wa2, ba2, Wf, Wfs1,
           bfs1, Wfs2, bfs2, Wr1, br1, Wr2, br2, Wrx1, Wrx2, We1, be1, We2,
           be2, Wae1, bae1, Wae2, bae2, Wae3, bae3):
    B = Z.shape[0]
    tab = jnp.pad(R, ((0, 0), (0, 0), (0, NF - 3)))
    out = pl.pallas_call(
        _probe_body,
        grid=(B,),
        in_specs=[
            pl.BlockSpec((1, A, NF), lambda b: (b, 0, 0)),
            pl.BlockSpec((1, A, NB), lambda b: (b, 0, 0)),
        ],
        out_specs=pl.BlockSpec((1, 1), lambda b: (b, 0)),
        out_shape=jax.ShapeDtypeStruct((B, 1), jnp.float32),
    )(tab, N.astype(jnp.int32))
    return out


# speedup vs baseline: 2.1762x; 2.1762x over previous
import functools

import jax
import jax.numpy as jnp
from jax.experimental import pallas as pl
from jax.experimental.pallas import tpu as pltpu

A = 512
NB = 32
NF = 128
RES = 20
CUTOFF = 5.0
NI = 3
EC = 2048
AC = EC // NB
NCH = (A * NB) // EC
E = A * NB

_HI = jax.lax.Precision.HIGHEST


def _dot(x, y):
    return jnp.dot(x, y, precision=_HI)


def _swish(x):
    return x * jax.nn.sigmoid(x)


def _body(rp_ref, z_ref, nhi_ref, nlo_ref, nm_ref, am_ref, emb_ref,
          wrbf_ref, wa1_ref, ba1_ref, wa2_ref, ba2_ref, wf_ref,
          wfs1_ref, bfs1_ref, wfs2_ref, bfs2_ref,
          wr1_ref, br1_ref, wr2_ref, br2_ref,
          wrx1_ref, wrx2_ref, we1_ref, be1_ref, we2_ref, be2_ref,
          wae1_ref, bae1_ref, wae2_ref, bae2_ref, wae3_ref, bae3_ref,
          out_ref,
          idxm, rbfc, dvnm, a_s, am_s, pr_s, pe_s, r_a, r_b, fd_s):
    rp = rp_ref[0]

    z = z_ref[0]
    zoh = (z == jax.lax.broadcasted_iota(jnp.int32, (A, 16), 1))
    a_s[...] = _dot(zoh.astype(jnp.float32), emb_ref[...])
    r_a[...] = jnp.zeros((A, 3 * NF), jnp.float32)
    fd_s[...] = jnp.zeros((A, 3 * NF), jnp.float32)

    def _geo(c, _):
        e0 = c * EC
        a0 = c * AC
        io_e = jax.lax.broadcasted_iota(jnp.int32, (EC, A), 0)
        io_s = jax.lax.broadcasted_iota(jnp.int32, (EC, A), 1)
        own = ((io_e // NB) + a0 == io_s).astype(jnp.float32)
        nhic = _dot(own, nhi_ref[0])
        nloc = _dot(own, nlo_ref[0])
        nmc = _dot(own, nm_ref[0])
        io_k0 = jax.lax.broadcasted_iota(jnp.int32, (EC, NB), 0)
        io_k1 = jax.lax.broadcasted_iota(jnp.int32, (EC, NB), 1)
        ksel = (io_k1 == io_k0 % NB).astype(jnp.float32)
        idxf = jnp.sum((nhic * 16.0 + nloc) * ksel, axis=1, keepdims=True)
        nmv = jnp.sum(nmc * ksel, axis=1, keepdims=True)
        idxv = jnp.where(nmv > 0.5, idxf, 9999.0).astype(jnp.int32)
        idxb = jnp.broadcast_to(idxv, (EC, NF))
        idxm[pl.ds(e0, EC), :] = idxb
        io_l = jax.lax.broadcasted_iota(jnp.int32, (EC, NF), 1)
        g = jnp.concatenate(
            [(idxb == io_l + NF * gg).astype(jnp.float32) for gg in range(4)],
            axis=1)
        rn = _dot(g, rp)
        ri = _dot(own, rp)
        dv = rn - ri
        d2 = jnp.sum(dv * dv, axis=1, keepdims=True)
        dist = jnp.sqrt(d2 + 1e-12)
        dvn = dv / (dist + 1e-8)
        dvnm[pl.ds(e0, EC), :] = dvn[:, 0:8]
        x = dist * (1.0 / CUTOFF)
        x2 = x * x
        x4 = x2 * x2
        x9 = x4 * x4 * x
        cpoly = (1.0 - 55.0 * x9 + 99.0 * x9 * x - 45.0 * x9 * x2)
        cutc = cpoly * (x < 1.0).astype(jnp.float32)
        distb = jnp.broadcast_to(dist, (EC, NF))
        cutb = jnp.broadcast_to(cutc, (EC, NF))
        kf = (io_l + 1).astype(jnp.float32)
        sinv = jnp.sin(kf * (jnp.pi / CUTOFF) * distb) / (distb + 1e-8) * cutb
        rbf_eff = jnp.where(io_l == RES, cutb,
                            jnp.where(io_l < RES, sinv, 0.0))
        rbfc[pl.ds(e0, EC), :] = rbf_eff[:, 0:24]
        return _

    jax.lax.fori_loop(0, NCH, _geo, None)

    rcur, rnxt = r_a, r_b
    for t in range(NI):
        a = a_s[...]
        am_s[...] = _dot(_swish(_dot(a, wa1_ref[t]) + ba1_ref[t]),
                         wa2_ref[t]) + ba2_ref[t]
        pr_s[...] = _dot(_swish(_dot(a, wr1_ref[t]) + br1_ref[t]),
                         wr2_ref[t]) + br2_ref[t]
        pe_s[...] = _dot(_swish(_dot(a, we1_ref[t]) + be1_ref[t]),
                         we2_ref[t]) + be2_ref[t]

        def _chunk(c, _, t=t, rcur=rcur, rnxt=rnxt):
            e0 = c * EC
            a0 = c * AC
            idxb = idxm[pl.ds(e0, EC), :]
            io_l = jax.lax.broadcasted_iota(jnp.int32, (EC, NF), 1)
            g = jnp.concatenate(
                [(idxb == io_l + NF * gg).astype(jnp.float32)
                 for gg in range(4)], axis=1)
            rbf_m = _dot(rbfc[pl.ds(e0, EC), :], wrbf_ref[t])
            aj = _dot(g, am_s[...])
            ai = jnp.broadcast_to(
                am_s[pl.ds(a0, AC), :].reshape(AC, 1, NF),
                (AC, NB, NF)).reshape(EC, NF)
            msij = rbf_m * aj * ai
            fsc = _dot(msij, wf_ref[t])
            fs = _dot(_swish(_dot(msij, wfs1_ref[t]) + bfs1_ref[t]),
                      wfs2_ref[t]) + bfs2_ref[t]
            rx = _dot(_swish(_dot(msij, wrx1_ref[t])), wrx2_ref[t])
            drj = _dot(g, rcur[...])
            prc = pr_s[pl.ds(a0, AC), :]
            ff = fs * fsc
            de = jnp.zeros((AC, NF), jnp.float32)
            for d in range(3):
                dvd = dvnm[pl.ds(e0, EC), d:d + 1]
                f4 = ff * dvd
                fid = jnp.sum(f4.reshape(AC, NB, NF), axis=1)
                dre = jnp.sum(
                    (rx * drj[:, d * NF:(d + 1) * NF]).reshape(AC, NB, NF),
                    axis=1)
                rnew = rcur[pl.ds(a0, AC), d * NF:(d + 1) * NF] \
                    + prc * fid + dre
                fnew = fd_s[pl.ds(a0, AC), d * NF:(d + 1) * NF] + fid
                fd_s[pl.ds(a0, AC), d * NF:(d + 1) * NF] = fnew
                rnxt[pl.ds(a0, AC), d * NF:(d + 1) * NF] = rnew
                de = de + fnew * rnew
            pec = pe_s[pl.ds(a0, AC), :]
            a_s[pl.ds(a0, AC), :] = a_s[pl.ds(a0, AC), :] - pec * de
            return _

        jax.lax.fori_loop(0, NCH, _chunk, None)
        rcur, rnxt = rnxt, rcur

    a = a_s[...]
    h = _swish(_dot(a, wae1_ref[...]) + bae1_ref[...])
    h = _swish(_dot(h, wae2_ref[...]) + bae2_ref[...])
    ei = _dot(h, wae3_ref[...]) + bae3_ref[...]
    er = _dot(am_ref[0], ei)
    out_ref[0] = er[:, 0:1]


def kernel(Z, R, N, NM, AM, emb, Wrbf, brbf, Wa1, ba1, Wa2, ba2, Wf, Wfs1,
           bfs1, Wfs2, bfs2, Wr1, br1, Wr2, br2, Wrx1, Wrx2, We1, be1, We2,
           be2, Wae1, bae1, Wae2, bae2, Wae3, bae3):
    B = Z.shape[0]
    f32 = jnp.float32
    rp = jnp.pad(R.astype(f32), ((0, 0), (0, 0), (0, NF - 3)))
    zr = Z.astype(jnp.int32).reshape(B, A, 1)
    n32 = N.astype(jnp.int32)
    nhi = (n32 // 16).astype(f32)
    nlo = (n32 % 16).astype(f32)
    nmf = NM.astype(f32)
    amr = AM.astype(f32).reshape(B, 1, A)
    embp = jnp.pad(emb.astype(f32), ((0, 6), (0, 0)))
    wrbf = jnp.pad(
        jnp.concatenate([Wrbf, brbf[:, None, :]], axis=1).astype(f32),
        ((0, 0), (0, 24 - RES - 1), (0, 0)))
    wf_rep = jnp.broadcast_to(Wf.astype(f32), (NI, NF, NF))
    b2 = lambda b: b.astype(f32).reshape(NI, 1, NF)
    wae2 = jnp.pad(Wae2.astype(f32), ((0, 0), (0, NF - 64)))
    bae2 = jnp.pad(bae2.astype(f32), ((0, NF - 64))).reshape(1, NF)
    wae3 = jnp.pad(Wae3.astype(f32), ((0, NF - 64), (0, NF - 1)))
    bae3 = jnp.pad(bae3.astype(f32), ((0, NF - 1))).reshape(1, NF)
    bae1 = bae1.astype(f32).reshape(1, NF)

    wspec = lambda s: pl.BlockSpec(s, lambda b: (0,) * len(s))
    in_specs = [
            pl.BlockSpec((1, A, NF), lambda b: (b, 0, 0)),
            pl.BlockSpec((1, A, 1), lambda b: (b, 0, 0)),
            pl.BlockSpec((1, A, NB), lambda b: (b, 0, 0)),
            pl.BlockSpec((1, A, NB), lambda b: (b, 0, 0)),
            pl.BlockSpec((1, A, NB), lambda b: (b, 0, 0)),
            pl.BlockSpec((1, 1, A), lambda b: (b, 0, 0)),
            wspec((16, NF)),
            wspec((NI, 24, NF)),
            wspec((NI, NF, NF)), wspec((NI, 1, NF)),
            wspec((NI, NF, NF)), wspec((NI, 1, NF)),
            wspec((NI, NF, NF)),
            wspec((NI, NF, NF)), wspec((NI, 1, NF)),
            wspec((NI, NF, NF)), wspec((NI, 1, NF)),
            wspec((NI, NF, NF)), wspec((NI, 1, NF)),
            wspec((NI, NF, NF)), wspec((NI, 1, NF)),
            wspec((NI, NF, NF)), wspec((NI, NF, NF)),
            wspec((NI, NF, NF)), wspec((NI, 1, NF)),
            wspec((NI, NF, NF)), wspec((NI, 1, NF)),
            wspec((NF, NF)), wspec((1, NF)),
            wspec((NF, NF)), wspec((1, NF)),
            wspec((NF, NF)), wspec((1, NF)),
        ]
    out = pl.pallas_call(
        _body,
        grid=(B,),
        in_specs=in_specs,
        out_specs=pl.BlockSpec((1, 1, 1), lambda b: (b, 0, 0)),
        out_shape=jax.ShapeDtypeStruct((B, 1, 1), f32),
        scratch_shapes=[
            pltpu.VMEM((E, NF), jnp.int32),
            pltpu.VMEM((E, 24), f32),
            pltpu.VMEM((E, 8), f32),
            pltpu.VMEM((A, NF), f32),
            pltpu.VMEM((A, NF), f32),
            pltpu.VMEM((A, NF), f32),
            pltpu.VMEM((A, NF), f32),
            pltpu.VMEM((A, 3 * NF), f32),
            pltpu.VMEM((A, 3 * NF), f32),
            pltpu.VMEM((A, 3 * NF), f32),
        ],
    )(rp, zr, nhi, nlo, nmf, amr, embp, wrbf,
      Wa1.astype(f32), b2(ba1), Wa2.astype(f32), b2(ba2), wf_rep,
      Wfs1.astype(f32), b2(bfs1), Wfs2.astype(f32), b2(bfs2),
      Wr1.astype(f32), b2(br1), Wr2.astype(f32), b2(br2),
      Wrx1.astype(f32), Wrx2.astype(f32),
      We1.astype(f32), b2(be1), We2.astype(f32), b2(be2),
      Wae1.astype(f32), bae1, wae2, bae2, wae3, bae3)
    return out.reshape(B, 1)


# speedup vs baseline: 5.4465x; 2.5028x over previous
import functools

import jax
import jax.numpy as jnp
from jax.experimental import pallas as pl
from jax.experimental.pallas import tpu as pltpu

A = 512
NB = 32
NF = 128
RES = 20
CUTOFF = 5.0
NI = 3
EC = 2048
AC = EC // NB
NCH = (A * NB) // EC
E = A * NB

_HI = jax.lax.Precision.HIGHEST


def _dot(x, y):
    return jnp.dot(x, y, precision=_HI)


def _dotd(x, y):
    return jnp.dot(x, y, preferred_element_type=jnp.float32)


def _gather(g16, table):
    hi = table.astype(jnp.bfloat16)
    lo = (table - hi.astype(jnp.float32)).astype(jnp.bfloat16)
    return _dotd(g16, hi) + _dotd(g16, lo)


def _swish(x):
    return x * jax.nn.sigmoid(x)


def _body(rp_ref, z_ref, nhi_ref, nlo_ref, nm_ref, am_ref, emb_ref,
          wrbf_ref, wa1_ref, ba1_ref, wa2_ref, ba2_ref, wf_ref,
          wfs1_ref, bfs1_ref, wfs2_ref, bfs2_ref,
          wr1_ref, br1_ref, wr2_ref, br2_ref,
          wrx1_ref, wrx2_ref, we1_ref, be1_ref, we2_ref, be2_ref,
          wae1_ref, bae1_ref, wae2_ref, bae2_ref, wae3_ref, bae3_ref,
          out_ref,
          idxm, rbfc, dvnm, a_s, am_s, pr_s, pe_s, r_a, r_b, fd_s):
    rp = rp_ref[0]

    z = z_ref[0]
    zoh = (z == jax.lax.broadcasted_iota(jnp.int32, (A, 16), 1))
    a_s[...] = _dot(zoh.astype(jnp.float32), emb_ref[...])
    r_a[...] = jnp.zeros((A, 3 * NF), jnp.float32)
    fd_s[...] = jnp.zeros((A, 3 * NF), jnp.float32)

    def _geo(c, _):
        e0 = c * EC
        a0 = c * AC
        io_e = jax.lax.broadcasted_iota(jnp.int32, (EC, A), 0)
        io_s = jax.lax.broadcasted_iota(jnp.int32, (EC, A), 1)
        own = ((io_e // NB) + a0 == io_s).astype(jnp.float32)
        nhic = _dot(own, nhi_ref[0])
        nloc = _dot(own, nlo_ref[0])
        nmc = _dot(own, nm_ref[0])
        io_k0 = jax.lax.broadcasted_iota(jnp.int32, (EC, NB), 0)
        io_k1 = jax.lax.broadcasted_iota(jnp.int32, (EC, NB), 1)
        ksel = (io_k1 == io_k0 % NB).astype(jnp.float32)
        idxf = jnp.sum((nhic * 16.0 + nloc) * ksel, axis=1, keepdims=True)
        nmv = jnp.sum(nmc * ksel, axis=1, keepdims=True)
        idxv = jnp.where(nmv > 0.5, idxf, 9999.0).astype(jnp.int32)
        idxb = jnp.broadcast_to(idxv, (EC, NF))
        idxm[pl.ds(e0, EC), :] = idxb
        io_l = jax.lax.broadcasted_iota(jnp.int32, (EC, NF), 1)
        g = jnp.concatenate(
            [(idxb == io_l + NF * gg).astype(jnp.float32) for gg in range(4)],
            axis=1)
        rn = _dot(g, rp)
        ri = _dot(own, rp)
        dv = rn - ri
        d2 = jnp.sum(dv * dv, axis=1, keepdims=True)
        dist = jnp.sqrt(d2 + 1e-12)
        dvn = dv / (dist + 1e-8)
        dvnm[pl.ds(e0, EC), :] = dvn[:, 0:8]
        x = dist * (1.0 / CUTOFF)
        x2 = x * x
        x4 = x2 * x2
        x9 = x4 * x4 * x
        cpoly = (1.0 - 55.0 * x9 + 99.0 * x9 * x - 45.0 * x9 * x2)
        cutc = cpoly * (x < 1.0).astype(jnp.float32)
        distb = jnp.broadcast_to(dist, (EC, NF))
        cutb = jnp.broadcast_to(cutc, (EC, NF))
        kf = (io_l + 1).astype(jnp.float32)
        sinv = jnp.sin(kf * (jnp.pi / CUTOFF) * distb) / (distb + 1e-8) * cutb
        rbf_eff = jnp.where(io_l == RES, cutb,
                            jnp.where(io_l < RES, sinv, 0.0))
        rbfc[pl.ds(e0, EC), :] = rbf_eff[:, 0:24]
        return _

    jax.lax.fori_loop(0, NCH, _geo, None)

    rcur, rnxt = r_a, r_b
    for t in range(NI):
        a = a_s[...]
        am_s[...] = _dot(_swish(_dot(a, wa1_ref[t]) + ba1_ref[t]),
                         wa2_ref[t]) + ba2_ref[t]
        pr_s[...] = _dot(_swish(_dot(a, wr1_ref[t]) + br1_ref[t]),
                         wr2_ref[t]) + br2_ref[t]
        pe_s[...] = _dot(_swish(_dot(a, we1_ref[t]) + be1_ref[t]),
                         we2_ref[t]) + be2_ref[t]

        def _chunk(c, _, t=t, rcur=rcur, rnxt=rnxt):
            e0 = c * EC
            a0 = c * AC
            idxb = idxm[pl.ds(e0, EC), :]
            io_l = jax.lax.broadcasted_iota(jnp.int32, (EC, NF), 1)
            g16 = jnp.concatenate(
                [(idxb == io_l + NF * gg).astype(jnp.bfloat16)
                 for gg in range(4)], axis=1)
            rbf_m = _dotd(rbfc[pl.ds(e0, EC), :], wrbf_ref[t])
            gath = _gather(g16, jnp.concatenate([am_s[...], rcur[...]],
                                                axis=1))
            aj = gath[:, 0:NF]
            ai = jnp.broadcast_to(
                am_s[pl.ds(a0, AC), :].reshape(AC, 1, NF),
                (AC, NB, NF)).reshape(EC, NF)
            msij = rbf_m * aj * ai
            fsc = _dotd(msij, wf_ref[t])
            fs = _dotd(_swish(_dotd(msij, wfs1_ref[t]) + bfs1_ref[t]),
                       wfs2_ref[t]) + bfs2_ref[t]
            rx = _dotd(_swish(_dotd(msij, wrx1_ref[t])), wrx2_ref[t])
            drj = gath[:, NF:]
            prc = pr_s[pl.ds(a0, AC), :]
            ff = fs * fsc
            de = jnp.zeros((AC, NF), jnp.float32)
            for d in range(3):
                dvd = dvnm[pl.ds(e0, EC), d:d + 1]
                f4 = ff * dvd
                fid = jnp.sum(f4.reshape(AC, NB, NF), axis=1)
                dre = jnp.sum(
                    (rx * drj[:, d * NF:(d + 1) * NF]).reshape(AC, NB, NF),
                    axis=1)
                rnew = rcur[pl.ds(a0, AC), d * NF:(d + 1) * NF] \
                    + prc * fid + dre
                fnew = fd_s[pl.ds(a0, AC), d * NF:(d + 1) * NF] + fid
                fd_s[pl.ds(a0, AC), d * NF:(d + 1) * NF] = fnew
                rnxt[pl.ds(a0, AC), d * NF:(d + 1) * NF] = rnew
                de = de + fnew * rnew
            pec = pe_s[pl.ds(a0, AC), :]
            a_s[pl.ds(a0, AC), :] = a_s[pl.ds(a0, AC), :] - pec * de
            return _

        jax.lax.fori_loop(0, NCH, _chunk, None)
        rcur, rnxt = rnxt, rcur

    a = a_s[...]
    h = _swish(_dot(a, wae1_ref[...]) + bae1_ref[...])
    h = _swish(_dot(h, wae2_ref[...]) + bae2_ref[...])
    ei = _dot(h, wae3_ref[...]) + bae3_ref[...]
    er = _dot(am_ref[0], ei)
    out_ref[0] = er[:, 0:1]


def kernel(Z, R, N, NM, AM, emb, Wrbf, brbf, Wa1, ba1, Wa2, ba2, Wf, Wfs1,
           bfs1, Wfs2, bfs2, Wr1, br1, Wr2, br2, Wrx1, Wrx2, We1, be1, We2,
           be2, Wae1, bae1, Wae2, bae2, Wae3, bae3):
    B = Z.shape[0]
    f32 = jnp.float32
    rp = jnp.pad(R.astype(f32), ((0, 0), (0, 0), (0, NF - 3)))
    zr = Z.astype(jnp.int32).reshape(B, A, 1)
    n32 = N.astype(jnp.int32)
    nhi = (n32 // 16).astype(f32)
    nlo = (n32 % 16).astype(f32)
    nmf = NM.astype(f32)
    amr = AM.astype(f32).reshape(B, 1, A)
    embp = jnp.pad(emb.astype(f32), ((0, 6), (0, 0)))
    wrbf = jnp.pad(
        jnp.concatenate([Wrbf, brbf[:, None, :]], axis=1).astype(f32),
        ((0, 0), (0, 24 - RES - 1), (0, 0)))
    wf_rep = jnp.broadcast_to(Wf.astype(f32), (NI, NF, NF))
    b2 = lambda b: b.astype(f32).reshape(NI, 1, NF)
    wae2 = jnp.pad(Wae2.astype(f32), ((0, 0), (0, NF - 64)))
    bae2 = jnp.pad(bae2.astype(f32), ((0, NF - 64))).reshape(1, NF)
    wae3 = jnp.pad(Wae3.astype(f32), ((0, NF - 64), (0, NF - 1)))
    bae3 = jnp.pad(bae3.astype(f32), ((0, NF - 1))).reshape(1, NF)
    bae1 = bae1.astype(f32).reshape(1, NF)

    wspec = lambda s: pl.BlockSpec(s, lambda b: (0,) * len(s))
    in_specs = [
            pl.BlockSpec((1, A, NF), lambda b: (b, 0, 0)),
            pl.BlockSpec((1, A, 1), lambda b: (b, 0, 0)),
            pl.BlockSpec((1, A, NB), lambda b: (b, 0, 0)),
            pl.BlockSpec((1, A, NB), lambda b: (b, 0, 0)),
            pl.BlockSpec((1, A, NB), lambda b: (b, 0, 0)),
            pl.BlockSpec((1, 1, A), lambda b: (b, 0, 0)),
            wspec((16, NF)),
            wspec((NI, 24, NF)),
            wspec((NI, NF, NF)), wspec((NI, 1, NF)),
            wspec((NI, NF, NF)), wspec((NI, 1, NF)),
            wspec((NI, NF, NF)),
            wspec((NI, NF, NF)), wspec((NI, 1, NF)),
            wspec((NI, NF, NF)), wspec((NI, 1, NF)),
            wspec((NI, NF, NF)), wspec((NI, 1, NF)),
            wspec((NI, NF, NF)), wspec((NI, 1, NF)),
            wspec((NI, NF, NF)), wspec((NI, NF, NF)),
            wspec((NI, NF, NF)), wspec((NI, 1, NF)),
            wspec((NI, NF, NF)), wspec((NI, 1, NF)),
            wspec((NF, NF)), wspec((1, NF)),
            wspec((NF, NF)), wspec((1, NF)),
            wspec((NF, NF)), wspec((1, NF)),
        ]
    out = pl.pallas_call(
        _body,
        grid=(B,),
        in_specs=in_specs,
        out_specs=pl.BlockSpec((1, 1, 1), lambda b: (b, 0, 0)),
        out_shape=jax.ShapeDtypeStruct((B, 1, 1), f32),
        scratch_shapes=[
            pltpu.VMEM((E, NF), jnp.int32),
            pltpu.VMEM((E, 24), f32),
            pltpu.VMEM((E, 8), f32),
            pltpu.VMEM((A, NF), f32),
            pltpu.VMEM((A, NF), f32),
            pltpu.VMEM((A, NF), f32),
            pltpu.VMEM((A, NF), f32),
            pltpu.VMEM((A, 3 * NF), f32),
            pltpu.VMEM((A, 3 * NF), f32),
            pltpu.VMEM((A, 3 * NF), f32),
        ],
    )(rp, zr, nhi, nlo, nmf, amr, embp, wrbf,
      Wa1.astype(f32), b2(ba1), Wa2.astype(f32), b2(ba2), wf_rep,
      Wfs1.astype(f32), b2(bfs1), Wfs2.astype(f32), b2(bfs2),
      Wr1.astype(f32), b2(br1), Wr2.astype(f32), b2(br2),
      Wrx1.astype(f32), Wrx2.astype(f32),
      We1.astype(f32), b2(be1), We2.astype(f32), b2(be2),
      Wae1.astype(f32), bae1, wae2, bae2, wae3, bae3)
    return out.reshape(B, 1)


# speedup vs baseline: 8.8020x; 1.6161x over previous
import functools

import jax
import jax.numpy as jnp
from jax.experimental import pallas as pl
from jax.experimental.pallas import tpu as pltpu

A = 512
NB = 32
NF = 128
RES = 20
CUTOFF = 5.0
NI = 3
EC = 2048
AC = EC // NB
NCH = (A * NB) // EC
E = A * NB

_HI = jax.lax.Precision.HIGHEST


def _dot(x, y):
    return jnp.dot(x, y, precision=_HI)


def _dotd(x, y):
    return jnp.dot(x, y, preferred_element_type=jnp.float32)


def _gather(g16, table):
    hi = table.astype(jnp.bfloat16)
    lo = (table - hi.astype(jnp.float32)).astype(jnp.bfloat16)
    return _dotd(g16, hi) + _dotd(g16, lo)


def _swish(x):
    return x * jax.nn.sigmoid(x)


def _body(rphi_ref, rplo_ref, z_ref, ngeo_ref, am_ref, emb_ref,
          wrbf_ref, wa1_ref, ba1_ref, wa2_ref, ba2_ref, wf_ref,
          wfs1_ref, bfs1_ref, wfs2_ref, bfs2_ref,
          wr1_ref, br1_ref, wr2_ref, br2_ref,
          wrx1_ref, wrx2_ref, we1_ref, be1_ref, we2_ref, be2_ref,
          wae1_ref, bae1_ref, wae2_ref, bae2_ref, wae3_ref, bae3_ref,
          out_ref,
          gsc, rbfc, dvnm, a_s, am_s, pr_s, pe_s, r_a, r_b, fd_s):
    rphi = rphi_ref[0]
    rplo = rplo_ref[0]

    z = z_ref[0]
    zoh = (z == jax.lax.broadcasted_iota(jnp.int32, (A, 16), 1))
    a_s[...] = _dot(zoh.astype(jnp.float32), emb_ref[...])
    r_a[...] = jnp.zeros((A, 3 * NF), jnp.float32)
    fd_s[...] = jnp.zeros((A, 3 * NF), jnp.float32)

    def _geo(c, _):
        e0 = c * EC
        a0 = c * AC
        io_e = jax.lax.broadcasted_iota(jnp.int32, (EC, A), 0)
        io_s = jax.lax.broadcasted_iota(jnp.int32, (EC, A), 1)
        own = ((io_e // NB) + a0 == io_s).astype(jnp.bfloat16)
        ngc = _dotd(own, ngeo_ref[0])
        io_k0 = jax.lax.broadcasted_iota(jnp.int32, (EC, NB), 0)
        io_k1 = jax.lax.broadcasted_iota(jnp.int32, (EC, NB), 1)
        ksel = (io_k1 == io_k0 % NB).astype(jnp.float32)
        idxf = jnp.sum((ngc[:, 0:NB] * 16.0 + ngc[:, NB:2 * NB]) * ksel,
                       axis=1, keepdims=True)
        nmv = jnp.sum(ngc[:, 2 * NB:3 * NB] * ksel, axis=1, keepdims=True)
        idxv = jnp.where(nmv > 0.5, idxf, 9999.0).astype(jnp.int32)
        idxb = jnp.broadcast_to(idxv, (EC, NF))
        io_l = jax.lax.broadcasted_iota(jnp.int32, (EC, NF), 1)
        g16 = jnp.concatenate(
            [(idxb == io_l + NF * gg).astype(jnp.bfloat16) for gg in range(4)],
            axis=1)
        gsc[pl.ds(e0, EC), :] = g16
        rn = _dotd(g16, rphi) + _dotd(g16, rplo)
        ri = _dotd(own, rphi) + _dotd(own, rplo)
        dv = rn - ri
        d2 = jnp.sum(dv * dv, axis=1, keepdims=True)
        dist = jnp.sqrt(d2 + 1e-12)
        dvn = dv / (dist + 1e-8)
        dvnm[pl.ds(e0, EC), :] = dvn[:, 0:8]
        x = dist * (1.0 / CUTOFF)
        x2 = x * x
        x4 = x2 * x2
        x9 = x4 * x4 * x
        cpoly = (1.0 - 55.0 * x9 + 99.0 * x9 * x - 45.0 * x9 * x2)
        cutc = cpoly * (x < 1.0).astype(jnp.float32)
        distb = jnp.broadcast_to(dist, (EC, NF))
        cutb = jnp.broadcast_to(cutc, (EC, NF))
        kf = (io_l + 1).astype(jnp.float32)
        sinv = jnp.sin(kf * (jnp.pi / CUTOFF) * distb) / (distb + 1e-8) * cutb
        rbf_eff = jnp.where(io_l == RES, cutb,
                            jnp.where(io_l < RES, sinv, 0.0))
        rbfc[pl.ds(e0, EC), :] = rbf_eff[:, 0:24]
        return _

    jax.lax.fori_loop(0, NCH, _geo, None)

    rcur, rnxt = r_a, r_b
    for t in range(NI):
        a = a_s[...]
        am_s[...] = _dot(_swish(_dot(a, wa1_ref[t]) + ba1_ref[t]),
                         wa2_ref[t]) + ba2_ref[t]
        pr_s[...] = _dot(_swish(_dot(a, wr1_ref[t]) + br1_ref[t]),
                         wr2_ref[t]) + br2_ref[t]
        pe_s[...] = _dot(_swish(_dot(a, we1_ref[t]) + be1_ref[t]),
                         we2_ref[t]) + be2_ref[t]

        def _chunk(c, _, t=t, rcur=rcur, rnxt=rnxt):
            e0 = c * EC
            a0 = c * AC
            g16 = gsc[pl.ds(e0, EC), :]
            rbf_m = _dotd(rbfc[pl.ds(e0, EC), :], wrbf_ref[t])
            gath = _gather(g16, jnp.concatenate([am_s[...], rcur[...]],
                                                axis=1))
            aj = gath[:, 0:NF]
            ai = jnp.broadcast_to(
                am_s[pl.ds(a0, AC), :].reshape(AC, 1, NF),
                (AC, NB, NF)).reshape(EC, NF)
            msij = rbf_m * aj * ai
            fsc = _dotd(msij, wf_ref[t])
            fs = _dotd(_swish(_dotd(msij, wfs1_ref[t]) + bfs1_ref[t]),
                       wfs2_ref[t]) + bfs2_ref[t]
            rx = _dotd(_swish(_dotd(msij, wrx1_ref[t])), wrx2_ref[t])
            drj = gath[:, NF:]
            prc = pr_s[pl.ds(a0, AC), :]
            ff = fs * fsc
            de = jnp.zeros((AC, NF), jnp.float32)
            for d in range(3):
                dvd = dvnm[pl.ds(e0, EC), d:d + 1]
                f4 = ff * dvd
                fid = jnp.sum(f4.reshape(AC, NB, NF), axis=1)
                dre = jnp.sum(
                    (rx * drj[:, d * NF:(d + 1) * NF]).reshape(AC, NB, NF),
                    axis=1)
                rnew = rcur[pl.ds(a0, AC), d * NF:(d + 1) * NF] \
                    + prc * fid + dre
                fnew = fd_s[pl.ds(a0, AC), d * NF:(d + 1) * NF] + fid
                fd_s[pl.ds(a0, AC), d * NF:(d + 1) * NF] = fnew
                rnxt[pl.ds(a0, AC), d * NF:(d + 1) * NF] = rnew
                de = de + fnew * rnew
            pec = pe_s[pl.ds(a0, AC), :]
            a_s[pl.ds(a0, AC), :] = a_s[pl.ds(a0, AC), :] - pec * de
            return _

        jax.lax.fori_loop(0, NCH, _chunk, None)
        rcur, rnxt = rnxt, rcur

    a = a_s[...]
    h = _swish(_dot(a, wae1_ref[...]) + bae1_ref[...])
    h = _swish(_dot(h, wae2_ref[...]) + bae2_ref[...])
    ei = _dot(h, wae3_ref[...]) + bae3_ref[...]
    er = _dot(am_ref[0], ei)
    out_ref[0] = er[:, 0:1]


def kernel(Z, R, N, NM, AM, emb, Wrbf, brbf, Wa1, ba1, Wa2, ba2, Wf, Wfs1,
           bfs1, Wfs2, bfs2, Wr1, br1, Wr2, br2, Wrx1, Wrx2, We1, be1, We2,
           be2, Wae1, bae1, Wae2, bae2, Wae3, bae3):
    B = Z.shape[0]
    f32 = jnp.float32
    rp = jnp.pad(R.astype(f32), ((0, 0), (0, 0), (0, NF - 3)))
    rphi = rp.astype(jnp.bfloat16)
    rplo = (rp - rphi.astype(f32)).astype(jnp.bfloat16)
    zr = Z.astype(jnp.int32).reshape(B, A, 1)
    n32 = N.astype(jnp.int32)
    ngeo = jnp.concatenate(
        [(n32 // 16).astype(f32), (n32 % 16).astype(f32), NM.astype(f32)],
        axis=2).astype(jnp.bfloat16)
    amr = AM.astype(f32).reshape(B, 1, A)
    embp = jnp.pad(emb.astype(f32), ((0, 6), (0, 0)))
    wrbf = jnp.pad(
        jnp.concatenate([Wrbf, brbf[:, None, :]], axis=1).astype(f32),
        ((0, 0), (0, 24 - RES - 1), (0, 0)))
    wf_rep = jnp.broadcast_to(Wf.astype(f32), (NI, NF, NF))
    b2 = lambda b: b.astype(f32).reshape(NI, 1, NF)
    wae2 = jnp.pad(Wae2.astype(f32), ((0, 0), (0, NF - 64)))
    bae2 = jnp.pad(bae2.astype(f32), ((0, NF - 64))).reshape(1, NF)
    wae3 = jnp.pad(Wae3.astype(f32), ((0, NF - 64), (0, NF - 1)))
    bae3 = jnp.pad(bae3.astype(f32), ((0, NF - 1))).reshape(1, NF)
    bae1 = bae1.astype(f32).reshape(1, NF)

    wspec = lambda s: pl.BlockSpec(s, lambda b: (0,) * len(s))
    in_specs = [
            pl.BlockSpec((1, A, NF), lambda b: (b, 0, 0)),
            pl.BlockSpec((1, A, NF), lambda b: (b, 0, 0)),
            pl.BlockSpec((1, A, 1), lambda b: (b, 0, 0)),
            pl.BlockSpec((1, A, 3 * NB), lambda b: (b, 0, 0)),
            pl.BlockSpec((1, 1, A), lambda b: (b, 0, 0)),
            wspec((16, NF)),
            wspec((NI, 24, NF)),
            wspec((NI, NF, NF)), wspec((NI, 1, NF)),
            wspec((NI, NF, NF)), wspec((NI, 1, NF)),
            wspec((NI, NF, NF)),
            wspec((NI, NF, NF)), wspec((NI, 1, NF)),
            wspec((NI, NF, NF)), wspec((NI, 1, NF)),
            wspec((NI, NF, NF)), wspec((NI, 1, NF)),
            wspec((NI, NF, NF)), wspec((NI, 1, NF)),
            wspec((NI, NF, NF)), wspec((NI, NF, NF)),
            wspec((NI, NF, NF)), wspec((NI, 1, NF)),
            wspec((NI, NF, NF)), wspec((NI, 1, NF)),
            wspec((NF, NF)), wspec((1, NF)),
            wspec((NF, NF)), wspec((1, NF)),
            wspec((NF, NF)), wspec((1, NF)),
        ]
    out = pl.pallas_call(
        _body,
        grid=(B,),
        in_specs=in_specs,
        out_specs=pl.BlockSpec((1, 1, 1), lambda b: (b, 0, 0)),
        out_shape=jax.ShapeDtypeStruct((B, 1, 1), f32),
        scratch_shapes=[
            pltpu.VMEM((E, A), jnp.bfloat16),
            pltpu.VMEM((E, 24), f32),
            pltpu.VMEM((E, 8), f32),
            pltpu.VMEM((A, NF), f32),
            pltpu.VMEM((A, NF), f32),
            pltpu.VMEM((A, NF), f32),
            pltpu.VMEM((A, NF), f32),
            pltpu.VMEM((A, 3 * NF), f32),
            pltpu.VMEM((A, 3 * NF), f32),
            pltpu.VMEM((A, 3 * NF), f32),
        ],
    )(rphi, rplo, zr, ngeo, amr, embp, wrbf,
      Wa1.astype(f32), b2(ba1), Wa2.astype(f32), b2(ba2), wf_rep,
      Wfs1.astype(f32), b2(bfs1), Wfs2.astype(f32), b2(bfs2),
      Wr1.astype(f32), b2(br1), Wr2.astype(f32), b2(br2),
      Wrx1.astype(f32), Wrx2.astype(f32),
      We1.astype(f32), b2(be1), We2.astype(f32), b2(be2),
      Wae1.astype(f32), bae1, wae2, bae2, wae3, bae3)
    return out.reshape(B, 1)


# speedup vs baseline: 10.7216x; 1.2181x over previous
import functools

import jax
import jax.numpy as jnp
from jax.experimental import pallas as pl
from jax.experimental.pallas import tpu as pltpu

A = 512
NB = 32
NF = 128
RES = 20
CUTOFF = 5.0
NI = 3
EC = 2048
AC = EC // NB
NCH = (A * NB) // EC
E = A * NB

_HI = jax.lax.Precision.HIGHEST


def _dot(x, y):
    return jnp.dot(x, y, precision=_HI)


def _dotd(x, y):
    return jnp.dot(x, y, preferred_element_type=jnp.float32)


def _dotb(x, y):
    return _dotd(x.astype(jnp.bfloat16), y.astype(jnp.bfloat16))


def _gather(g16, table):
    hi = table.astype(jnp.bfloat16)
    lo = (table - hi.astype(jnp.float32)).astype(jnp.bfloat16)
    return _dotd(g16, hi) + _dotd(g16, lo)


def _swish(x):
    return x * jax.nn.sigmoid(x)


def _body(rphi_ref, rplo_ref, z_ref, ngeo_ref, am_ref, emb_ref,
          wrbf_ref, wa1_ref, ba1_ref, wa2_ref, ba2_ref, wf_ref,
          wfs1_ref, bfs1_ref, wfs2_ref, bfs2_ref,
          wr1_ref, br1_ref, wr2_ref, br2_ref,
          wrx1_ref, wrx2_ref, we1_ref, be1_ref, we2_ref, be2_ref,
          wae1_ref, bae1_ref, wae2_ref, bae2_ref, wae3_ref, bae3_ref,
          out_ref,
          gsc, rbfc, dvnm, a_s, am_s, pr_s, pe_s, r_a, r_b, fd_s):
    rphi = rphi_ref[0]
    rplo = rplo_ref[0]

    z = z_ref[0]
    zoh = (z == jax.lax.broadcasted_iota(jnp.int32, (A, 16), 1))
    a_s[...] = _dot(zoh.astype(jnp.float32), emb_ref[...])
    r_a[...] = jnp.zeros((A, 3 * NF), jnp.float32)
    fd_s[...] = jnp.zeros((A, 3 * NF), jnp.float32)

    def _geo(c, _):
        e0 = c * EC
        a0 = c * AC
        io_e = jax.lax.broadcasted_iota(jnp.int32, (EC, A), 0)
        io_s = jax.lax.broadcasted_iota(jnp.int32, (EC, A), 1)
        own = ((io_e // NB) + a0 == io_s).astype(jnp.bfloat16)
        ngc = _dotd(own, ngeo_ref[0])
        io_k0 = jax.lax.broadcasted_iota(jnp.int32, (EC, NB), 0)
        io_k1 = jax.lax.broadcasted_iota(jnp.int32, (EC, NB), 1)
        ksel = (io_k1 == io_k0 % NB).astype(jnp.float32)
        idxf = jnp.sum((ngc[:, 0:NB] * 16.0 + ngc[:, NB:2 * NB]) * ksel,
                       axis=1, keepdims=True)
        nmv = jnp.sum(ngc[:, 2 * NB:3 * NB] * ksel, axis=1, keepdims=True)
        idxv = jnp.where(nmv > 0.5, idxf, 9999.0).astype(jnp.int32)
        idxb = jnp.broadcast_to(idxv, (EC, NF))
        io_l = jax.lax.broadcasted_iota(jnp.int32, (EC, NF), 1)
        g16 = jnp.concatenate(
            [(idxb == io_l + NF * gg).astype(jnp.bfloat16) for gg in range(4)],
            axis=1)
        gsc[pl.ds(e0, EC), :] = g16
        rn = _dotd(g16, rphi) + _dotd(g16, rplo)
        ri = _dotd(own, rphi) + _dotd(own, rplo)
        dv = rn - ri
        d2 = jnp.sum(dv * dv, axis=1, keepdims=True)
        dist = jnp.sqrt(d2 + 1e-12)
        dvn = dv / (dist + 1e-8)
        dvnm[pl.ds(e0, EC), :] = dvn[:, 0:8]
        x = dist * (1.0 / CUTOFF)
        x2 = x * x
        x4 = x2 * x2
        x9 = x4 * x4 * x
        cpoly = (1.0 - 55.0 * x9 + 99.0 * x9 * x - 45.0 * x9 * x2)
        cutc = cpoly * (x < 1.0).astype(jnp.float32)
        distb = jnp.broadcast_to(dist, (EC, NF))
        cutb = jnp.broadcast_to(cutc, (EC, NF))
        kf = (io_l + 1).astype(jnp.float32)
        u = distb * (kf * (1.0 / CUTOFF))
        m = jnp.floor(u + 0.5)
        sgn = jnp.where((m.astype(jnp.int32) % 2) == 0, 1.0, -1.0)
        xs = (u - m) * jnp.pi
        xq = xs * xs
        sinp = xs * (1.0 + xq * (-1.0 / 6.0 + xq * (1.0 / 120.0 + xq * (
            -1.0 / 5040.0 + xq * (1.0 / 362880.0)))))
        sinv = sgn * sinp / (distb + 1e-8) * cutb
        rbf_eff = jnp.where(io_l == RES, cutb,
                            jnp.where(io_l < RES, sinv, 0.0))
        rbfc[pl.ds(e0, EC), :] = rbf_eff[:, 0:24]
        return _

    jax.lax.fori_loop(0, NCH, _geo, None)

    rcur, rnxt = r_a, r_b
    for t in range(NI):
        a = a_s[...]
        am_s[...] = _dotb(_swish(_dotb(a, wa1_ref[t]) + ba1_ref[t]),
                          wa2_ref[t]) + ba2_ref[t]
        pr_s[...] = _dotb(_swish(_dotb(a, wr1_ref[t]) + br1_ref[t]),
                          wr2_ref[t]) + br2_ref[t]
        pe_s[...] = _dotb(_swish(_dotb(a, we1_ref[t]) + be1_ref[t]),
                          we2_ref[t]) + be2_ref[t]

        def _chunk(c, _, t=t, rcur=rcur, rnxt=rnxt):
            e0 = c * EC
            a0 = c * AC
            g16 = gsc[pl.ds(e0, EC), :]
            rbf_m = _dotb(rbfc[pl.ds(e0, EC), :], wrbf_ref[t])
            if t == 0:
                gath = _gather(g16, am_s[...])
                aj = gath
            else:
                gath = _gather(g16, jnp.concatenate([am_s[...], rcur[...]],
                                                    axis=1))
                aj = gath[:, 0:NF]
            ai3 = am_s[pl.ds(a0, AC), :].reshape(AC, 1, NF)
            msij = ((rbf_m * aj).reshape(AC, NB, NF) * ai3).reshape(EC, NF)
            fsc = _dotb(msij, wf_ref[t])
            fs = _dotb(_swish(_dotb(msij, wfs1_ref[t]) + bfs1_ref[t]),
                       wfs2_ref[t]) + bfs2_ref[t]
            if t > 0:
                rx = _dotb(_swish(_dotb(msij, wrx1_ref[t])), wrx2_ref[t])
            prc = pr_s[pl.ds(a0, AC), :]
            ff = fs * fsc
            de = jnp.zeros((AC, NF), jnp.float32)
            for d in range(3):
                dvd = dvnm[pl.ds(e0, EC), d:d + 1]
                f4 = ff * dvd
                fid = jnp.sum(f4.reshape(AC, NB, NF), axis=1)
                if t == 0:
                    rnew = prc * fid
                    fnew = fid
                else:
                    dre = jnp.sum(
                        (rx * gath[:, (d + 1) * NF:(d + 2) * NF]
                         ).reshape(AC, NB, NF), axis=1)
                    rnew = rcur[pl.ds(a0, AC), d * NF:(d + 1) * NF] \
                        + prc * fid + dre
                    fnew = fd_s[pl.ds(a0, AC), d * NF:(d + 1) * NF] + fid
                fd_s[pl.ds(a0, AC), d * NF:(d + 1) * NF] = fnew
                rnxt[pl.ds(a0, AC), d * NF:(d + 1) * NF] = rnew
                de = de + fnew * rnew
            pec = pe_s[pl.ds(a0, AC), :]
            a_s[pl.ds(a0, AC), :] = a_s[pl.ds(a0, AC), :] - pec * de
            return _

        jax.lax.fori_loop(0, NCH, _chunk, None)
        rcur, rnxt = rnxt, rcur

    a = a_s[...]
    h = _swish(_dotb(a, wae1_ref[...]) + bae1_ref[...])
    h = _swish(_dotb(h, wae2_ref[...]) + bae2_ref[...])
    ei = _dotb(h, wae3_ref[...]) + bae3_ref[...]
    er = _dot(am_ref[0], ei)
    out_ref[0] = er[:, 0:1]


def kernel(Z, R, N, NM, AM, emb, Wrbf, brbf, Wa1, ba1, Wa2, ba2, Wf, Wfs1,
           bfs1, Wfs2, bfs2, Wr1, br1, Wr2, br2, Wrx1, Wrx2, We1, be1, We2,
           be2, Wae1, bae1, Wae2, bae2, Wae3, bae3):
    B = Z.shape[0]
    f32 = jnp.float32
    rp = jnp.pad(R.astype(f32), ((0, 0), (0, 0), (0, NF - 3)))
    rphi = rp.astype(jnp.bfloat16)
    rplo = (rp - rphi.astype(f32)).astype(jnp.bfloat16)
    zr = Z.astype(jnp.int32).reshape(B, A, 1)
    n32 = N.astype(jnp.int32)
    ngeo = jnp.concatenate(
        [(n32 // 16).astype(f32), (n32 % 16).astype(f32), NM.astype(f32)],
        axis=2).astype(jnp.bfloat16)
    amr = AM.astype(f32).reshape(B, 1, A)
    embp = jnp.pad(emb.astype(f32), ((0, 6), (0, 0)))
    wrbf = jnp.pad(
        jnp.concatenate([Wrbf, brbf[:, None, :]], axis=1).astype(f32),
        ((0, 0), (0, 24 - RES - 1), (0, 0)))
    wf_rep = jnp.broadcast_to(Wf.astype(f32), (NI, NF, NF))
    b2 = lambda b: b.astype(f32).reshape(NI, 1, NF)
    wae2 = jnp.pad(Wae2.astype(f32), ((0, 0), (0, NF - 64)))
    bae2 = jnp.pad(bae2.astype(f32), ((0, NF - 64))).reshape(1, NF)
    wae3 = jnp.pad(Wae3.astype(f32), ((0, NF - 64), (0, NF - 1)))
    bae3 = jnp.pad(bae3.astype(f32), ((0, NF - 1))).reshape(1, NF)
    bae1 = bae1.astype(f32).reshape(1, NF)

    wspec = lambda s: pl.BlockSpec(s, lambda b: (0,) * len(s))
    in_specs = [
            pl.BlockSpec((1, A, NF), lambda b: (b, 0, 0)),
            pl.BlockSpec((1, A, NF), lambda b: (b, 0, 0)),
            pl.BlockSpec((1, A, 1), lambda b: (b, 0, 0)),
            pl.BlockSpec((1, A, 3 * NB), lambda b: (b, 0, 0)),
            pl.BlockSpec((1, 1, A), lambda b: (b, 0, 0)),
            wspec((16, NF)),
            wspec((NI, 24, NF)),
            wspec((NI, NF, NF)), wspec((NI, 1, NF)),
            wspec((NI, NF, NF)), wspec((NI, 1, NF)),
            wspec((NI, NF, NF)),
            wspec((NI, NF, NF)), wspec((NI, 1, NF)),
            wspec((NI, NF, NF)), wspec((NI, 1, NF)),
            wspec((NI, NF, NF)), wspec((NI, 1, NF)),
            wspec((NI, NF, NF)), wspec((NI, 1, NF)),
            wspec((NI, NF, NF)), wspec((NI, NF, NF)),
            wspec((NI, NF, NF)), wspec((NI, 1, NF)),
            wspec((NI, NF, NF)), wspec((NI, 1, NF)),
            wspec((NF, NF)), wspec((1, NF)),
            wspec((NF, NF)), wspec((1, NF)),
            wspec((NF, NF)), wspec((1, NF)),
        ]
    out = pl.pallas_call(
        _body,
        grid=(B,),
        in_specs=in_specs,
        out_specs=pl.BlockSpec((1, 1, 1), lambda b: (b, 0, 0)),
        out_shape=jax.ShapeDtypeStruct((B, 1, 1), f32),
        scratch_shapes=[
            pltpu.VMEM((E, A), jnp.bfloat16),
            pltpu.VMEM((E, 24), f32),
            pltpu.VMEM((E, 8), f32),
            pltpu.VMEM((A, NF), f32),
            pltpu.VMEM((A, NF), f32),
            pltpu.VMEM((A, NF), f32),
            pltpu.VMEM((A, NF), f32),
            pltpu.VMEM((A, 3 * NF), f32),
            pltpu.VMEM((A, 3 * NF), f32),
            pltpu.VMEM((A, 3 * NF), f32),
        ],
    )(rphi, rplo, zr, ngeo, amr, embp, wrbf,
      Wa1.astype(f32), b2(ba1), Wa2.astype(f32), b2(ba2), wf_rep,
      Wfs1.astype(f32), b2(bfs1), Wfs2.astype(f32), b2(bfs2),
      Wr1.astype(f32), b2(br1), Wr2.astype(f32), b2(br2),
      Wrx1.astype(f32), Wrx2.astype(f32),
      We1.astype(f32), b2(be1), We2.astype(f32), b2(be2),
      Wae1.astype(f32), bae1, wae2, bae2, wae3, bae3)
    return out.reshape(B, 1)


# speedup vs baseline: 10.7650x; 1.0041x over previous
import jax
import jax.numpy as jnp
from jax.experimental import pallas as pl
from jax.experimental.pallas import tpu as pltpu

A = 512
NB = 32
NF = 128
RES = 20
CUTOFF = 5.0
NI = 3
EC = 2048
AC = EC // NB
NCH = (A * NB) // EC
E = A * NB

_HI = jax.lax.Precision.HIGHEST


def _dot(x, y):
    return jnp.dot(x, y, precision=_HI)


def _dotd(x, y):
    return jnp.dot(x, y, preferred_element_type=jnp.float32)


def _dotb(x, y):
    return _dotd(x.astype(jnp.bfloat16), y.astype(jnp.bfloat16))


def _gather(g16, table):
    hi = table.astype(jnp.bfloat16)
    lo = (table - hi.astype(jnp.float32)).astype(jnp.bfloat16)
    return _dotd(g16, hi) + _dotd(g16, lo)


def _swish(x):
    return x * jax.nn.sigmoid(x)


def _body(rphi_ref, rplo_ref, z_ref, ngeo_ref, am_ref, emb_ref,
          wrbf_ref, wa1_ref, ba1_ref, wa2_ref, ba2_ref, wf_ref,
          wfs1_ref, bfs1_ref, wfs2_ref, bfs2_ref,
          wr1_ref, br1_ref, wr2_ref, br2_ref,
          wrx1_ref, wrx2_ref, we1_ref, be1_ref, we2_ref, be2_ref,
          wae1_ref, bae1_ref, wae2_ref, bae2_ref, wae3_ref, bae3_ref,
          out_ref,
          gsc, rbfc, dvnm, a_s, am_s, pr_s, pe_s, r_a, r_b, fd_s):
    rphi = rphi_ref[0]
    rplo = rplo_ref[0]

    z = z_ref[0]
    zoh = (z == jax.lax.broadcasted_iota(jnp.int32, (A, 16), 1))
    a_s[...] = _dot(zoh.astype(jnp.float32), emb_ref[...])
    r_a[...] = jnp.zeros((A, 3 * NF), jnp.float32)
    fd_s[...] = jnp.zeros((A, 3 * NF), jnp.float32)

    def _geo(c, _):
        e0 = c * EC
        a0 = c * AC
        io_e = jax.lax.broadcasted_iota(jnp.int32, (EC, A), 0)
        io_s = jax.lax.broadcasted_iota(jnp.int32, (EC, A), 1)
        own = ((io_e // NB) + a0 == io_s).astype(jnp.bfloat16)
        ngc = _dotd(own, ngeo_ref[0])
        io_k0 = jax.lax.broadcasted_iota(jnp.int32, (EC, NB), 0)
        io_k1 = jax.lax.broadcasted_iota(jnp.int32, (EC, NB), 1)
        ksel = (io_k1 == io_k0 % NB).astype(jnp.float32)
        idxf = jnp.sum((ngc[:, 0:NB] * 16.0 + ngc[:, NB:2 * NB]) * ksel,
                       axis=1, keepdims=True)
        nmv = jnp.sum(ngc[:, 2 * NB:3 * NB] * ksel, axis=1, keepdims=True)
        idxv = jnp.where(nmv > 0.5, idxf, 9999.0).astype(jnp.int32)
        idxb = jnp.broadcast_to(idxv, (EC, NF))
        io_l = jax.lax.broadcasted_iota(jnp.int32, (EC, NF), 1)
        g16 = jnp.concatenate(
            [(idxb == io_l + NF * gg).astype(jnp.bfloat16) for gg in range(4)],
            axis=1)
        gsc[pl.ds(e0, EC), :] = g16
        rn = _dotd(g16, rphi) + _dotd(g16, rplo)
        ri = jnp.broadcast_to(
            (rphi_ref[0, pl.ds(a0, AC), :].astype(jnp.float32)
             + rplo_ref[0, pl.ds(a0, AC), :].astype(jnp.float32)
             ).reshape(AC, 1, NF),
            (AC, NB, NF)).reshape(EC, NF)
        dv = rn - ri
        d2 = jnp.sum(dv * dv, axis=1, keepdims=True)
        dist = jnp.sqrt(d2 + 1e-12)
        dvn = dv / (dist + 1e-8)
        dvnm[pl.ds(e0, EC), :] = dvn[:, 0:8]
        x = dist * (1.0 / CUTOFF)
        x2 = x * x
        x4 = x2 * x2
        x9 = x4 * x4 * x
        cpoly = (1.0 - 55.0 * x9 + 99.0 * x9 * x - 45.0 * x9 * x2)
        cutc = cpoly * (x < 1.0).astype(jnp.float32)
        distb = jnp.broadcast_to(dist, (EC, NF))
        cutb = jnp.broadcast_to(cutc, (EC, NF))
        kf = (io_l + 1).astype(jnp.float32)
        u = distb * (kf * (1.0 / CUTOFF))
        m = jnp.floor(u + 0.5)
        sgn = jnp.where((m.astype(jnp.int32) % 2) == 0, 1.0, -1.0)
        xs = (u - m) * jnp.pi
        xq = xs * xs
        sinp = xs * (1.0 + xq * (-1.0 / 6.0 + xq * (1.0 / 120.0 + xq * (
            -1.0 / 5040.0 + xq * (1.0 / 362880.0)))))
        sinv = sgn * sinp / (distb + 1e-8) * cutb
        rbf_eff = jnp.where(io_l == RES, cutb,
                            jnp.where(io_l < RES, sinv, 0.0))
        rbfc[pl.ds(e0, EC), :] = rbf_eff[:, 0:24]
        return _

    jax.lax.fori_loop(0, NCH, _geo, None)

    rcur, rnxt = r_a, r_b
    for t in range(NI):
        a = a_s[...]
        am_s[...] = _dotb(_swish(_dotb(a, wa1_ref[t]) + ba1_ref[t]),
                          wa2_ref[t]) + ba2_ref[t]
        pr_s[...] = _dotb(_swish(_dotb(a, wr1_ref[t]) + br1_ref[t]),
                          wr2_ref[t]) + br2_ref[t]
        pe_s[...] = _dotb(_swish(_dotb(a, we1_ref[t]) + be1_ref[t]),
                          we2_ref[t]) + be2_ref[t]

        def _chunk(c, _, t=t, rcur=rcur, rnxt=rnxt):
            e0 = c * EC
            a0 = c * AC
            g16 = gsc[pl.ds(e0, EC), :]
            rbf_m = _dotb(rbfc[pl.ds(e0, EC), :], wrbf_ref[t])
            if t == 0:
                gath = _gather(g16, am_s[...])
                aj = gath
            else:
                gath = _gather(g16, jnp.concatenate([am_s[...], rcur[...]],
                                                    axis=1))
                aj = gath[:, 0:NF]
            ai3 = am_s[pl.ds(a0, AC), :].reshape(AC, 1, NF)
            msij = ((rbf_m * aj).reshape(AC, NB, NF) * ai3).reshape(EC, NF)
            fsc = _dotb(msij, wf_ref[t])
            fs = _dotb(_swish(_dotb(msij, wfs1_ref[t]) + bfs1_ref[t]),
                       wfs2_ref[t]) + bfs2_ref[t]
            if t > 0:
                rx = _dotb(_swish(_dotb(msij, wrx1_ref[t])), wrx2_ref[t])
            prc = pr_s[pl.ds(a0, AC), :]
            ff = fs * fsc
            de = jnp.zeros((AC, NF), jnp.float32)
            for d in range(3):
                dvd = dvnm[pl.ds(e0, EC), d:d + 1]
                f4 = ff * dvd
                fid = jnp.sum(f4.reshape(AC, NB, NF), axis=1)
                if t == 0:
                    rnew = prc * fid
                    fnew = fid
                else:
                    dre = jnp.sum(
                        (rx * gath[:, (d + 1) * NF:(d + 2) * NF]
                         ).reshape(AC, NB, NF), axis=1)
                    rnew = rcur[pl.ds(a0, AC), d * NF:(d + 1) * NF] \
                        + prc * fid + dre
                    fnew = fd_s[pl.ds(a0, AC), d * NF:(d + 1) * NF] + fid
                fd_s[pl.ds(a0, AC), d * NF:(d + 1) * NF] = fnew
                rnxt[pl.ds(a0, AC), d * NF:(d + 1) * NF] = rnew
                de = de + fnew * rnew
            pec = pe_s[pl.ds(a0, AC), :]
            a_s[pl.ds(a0, AC), :] = a_s[pl.ds(a0, AC), :] - pec * de
            return _

        jax.lax.fori_loop(0, NCH, _chunk, None)
        rcur, rnxt = rnxt, rcur

    a = a_s[...]
    h = _swish(_dotb(a, wae1_ref[...]) + bae1_ref[...])
    h = _swish(_dotb(h, wae2_ref[...]) + bae2_ref[...])
    ei = _dotb(h, wae3_ref[...]) + bae3_ref[...]
    er = _dot(am_ref[0], ei)
    out_ref[0] = er[:, 0:1]


def kernel(Z, R, N, NM, AM, emb, Wrbf, brbf, Wa1, ba1, Wa2, ba2, Wf, Wfs1,
           bfs1, Wfs2, bfs2, Wr1, br1, Wr2, br2, Wrx1, Wrx2, We1, be1, We2,
           be2, Wae1, bae1, Wae2, bae2, Wae3, bae3):
    B = Z.shape[0]
    f32 = jnp.float32
    rp = jnp.pad(R.astype(f32), ((0, 0), (0, 0), (0, NF - 3)))
    rphi = rp.astype(jnp.bfloat16)
    rplo = (rp - rphi.astype(f32)).astype(jnp.bfloat16)
    zr = Z.astype(jnp.int32).reshape(B, A, 1)
    n32 = N.astype(jnp.int32)
    ngeo = jnp.concatenate(
        [(n32 // 16).astype(f32), (n32 % 16).astype(f32), NM.astype(f32)],
        axis=2).astype(jnp.bfloat16)
    amr = AM.astype(f32).reshape(B, 1, A)
    embp = jnp.pad(emb.astype(f32), ((0, 6), (0, 0)))
    wrbf = jnp.pad(
        jnp.concatenate([Wrbf, brbf[:, None, :]], axis=1).astype(f32),
        ((0, 0), (0, 24 - RES - 1), (0, 0)))
    wf_rep = jnp.broadcast_to(Wf.astype(f32), (NI, NF, NF))
    b2 = lambda b: b.astype(f32).reshape(NI, 1, NF)
    wae2 = jnp.pad(Wae2.astype(f32), ((0, 0), (0, NF - 64)))
    bae2 = jnp.pad(bae2.astype(f32), ((0, NF - 64))).reshape(1, NF)
    wae3 = jnp.pad(Wae3.astype(f32), ((0, NF - 64), (0, NF - 1)))
    bae3 = jnp.pad(bae3.astype(f32), ((0, NF - 1))).reshape(1, NF)
    bae1 = bae1.astype(f32).reshape(1, NF)

    wspec = lambda s: pl.BlockSpec(s, lambda b: (0,) * len(s))
    in_specs = [
            pl.BlockSpec((1, A, NF), lambda b: (b, 0, 0)),
            pl.BlockSpec((1, A, NF), lambda b: (b, 0, 0)),
            pl.BlockSpec((1, A, 1), lambda b: (b, 0, 0)),
            pl.BlockSpec((1, A, 3 * NB), lambda b: (b, 0, 0)),
            pl.BlockSpec((1, 1, A), lambda b: (b, 0, 0)),
            wspec((16, NF)),
            wspec((NI, 24, NF)),
            wspec((NI, NF, NF)), wspec((NI, 1, NF)),
            wspec((NI, NF, NF)), wspec((NI, 1, NF)),
            wspec((NI, NF, NF)),
            wspec((NI, NF, NF)), wspec((NI, 1, NF)),
            wspec((NI, NF, NF)), wspec((NI, 1, NF)),
            wspec((NI, NF, NF)), wspec((NI, 1, NF)),
            wspec((NI, NF, NF)), wspec((NI, 1, NF)),
            wspec((NI, NF, NF)), wspec((NI, NF, NF)),
            wspec((NI, NF, NF)), wspec((NI, 1, NF)),
            wspec((NI, NF, NF)), wspec((NI, 1, NF)),
            wspec((NF, NF)), wspec((1, NF)),
            wspec((NF, NF)), wspec((1, NF)),
            wspec((NF, NF)), wspec((1, NF)),
        ]
    out = pl.pallas_call(
        _body,
        grid=(B,),
        in_specs=in_specs,
        out_specs=pl.BlockSpec((1, 1, 1), lambda b: (b, 0, 0)),
        out_shape=jax.ShapeDtypeStruct((B, 1, 1), f32),
        scratch_shapes=[
            pltpu.VMEM((E, A), jnp.bfloat16),
            pltpu.VMEM((E, 24), f32),
            pltpu.VMEM((E, 8), f32),
            pltpu.VMEM((A, NF), f32),
            pltpu.VMEM((A, NF), f32),
            pltpu.VMEM((A, NF), f32),
            pltpu.VMEM((A, NF), f32),
            pltpu.VMEM((A, 3 * NF), f32),
            pltpu.VMEM((A, 3 * NF), f32),
            pltpu.VMEM((A, 3 * NF), f32),
        ],
    )(rphi, rplo, zr, ngeo, amr, embp, wrbf,
      Wa1.astype(f32), b2(ba1), Wa2.astype(f32), b2(ba2), wf_rep,
      Wfs1.astype(f32), b2(bfs1), Wfs2.astype(f32), b2(bfs2),
      Wr1.astype(f32), b2(br1), Wr2.astype(f32), b2(br2),
      Wrx1.astype(f32), Wrx2.astype(f32),
      We1.astype(f32), b2(be1), We2.astype(f32), b2(be2),
      Wae1.astype(f32), bae1, wae2, bae2, wae3, bae3)
    return out.reshape(B, 1)
